# Initial kernel scaffold; baseline (speedup 1.0000x reference)
import jax
import jax.numpy as jnp
from jax.experimental import pallas as pl


def kernel(atoms, pos_amp, embed_table, pa_w, pa_b, reduce_w, reduce_b, stack_w, stack_b, head_atom_w, head_atom_b, head_pa_w, head_pa_b):
    raise NotImplementedError("write your pallas kernel here")


# speedup vs baseline: 1.9708x; 1.9708x over previous
import functools

import jax
import jax.numpy as jnp
from jax import lax
from jax.experimental import pallas as pl
from jax.experimental.pallas import tpu as pltpu
from jax.experimental.pallas import tpu_sc as plsc


def _sc_gather(table, idx):
    V, D = table.shape
    N = idx.shape[0]
    info = plsc.get_sparse_core_info()
    NC, NS = info.num_cores, info.num_subcores
    NW = NC * NS
    n_per_w = N // NW
    CHUNK = 256
    n_chunks = n_per_w // CHUNK
    mesh = plsc.VectorSubcoreMesh(core_axis_name="c", subcore_axis_name="s")

    @functools.partial(
        pl.kernel, mesh=mesh,
        out_type=jax.ShapeDtypeStruct((N, D), jnp.float32),
        scratch_types=[
            pltpu.VMEM((CHUNK,), jnp.int32),
            pltpu.VMEM((CHUNK, D), jnp.float32),
            pltpu.VMEM((CHUNK, D), jnp.float32),
            pltpu.SemaphoreType.DMA,
            pltpu.SemaphoreType.DMA,
        ],
    )
    def k(table_hbm, idx_hbm, out_hbm, idx_v, rows_a, rows_b, sem_a, sem_b):
        wid = lax.axis_index("s") * NC + lax.axis_index("c")
        base = wid * n_per_w
        bufs = ((rows_a, sem_a), (rows_b, sem_b))
        for j in range(n_chunks):
            rows_v, sem = bufs[j % 2]
            off = base + j * CHUNK
            pltpu.sync_copy(idx_hbm.at[pl.ds(off, CHUNK)], idx_v)
            pltpu.async_copy(table_hbm.at[idx_v], rows_v, sem).wait()
            pltpu.sync_copy(rows_v, out_hbm.at[pl.ds(off, CHUNK)])

    return k(table, idx)


def _tc_body(x_ref, pa_in_ref, pa_w_ref, pa_b_ref, wrx_ref, wrpa_ref,
             red_b_ref, stw0_ref, stw1_ref, stb_ref, hw_ref, hb_ref,
             hpw_ref, hpb_ref, out_ref, *, dilations):
    T, C = x_ref.shape[1], x_ref.shape[2]
    f32 = jnp.float32
    x = x_ref[0]
    pos_amp = pa_in_ref[0]
    pa = jnp.dot(pos_amp, pa_w_ref[...], preferred_element_type=f32) + pa_b_ref[...]
    h = (jnp.dot(x, wrx_ref[...], preferred_element_type=f32)
         + jnp.dot(pa, wrpa_ref[...], preferred_element_type=f32)
         + red_b_ref[...])
    for i, d in enumerate(dilations):
        h_shift = jnp.concatenate([jnp.zeros((d, C), f32), h[:T - d, :]], axis=0)
        z = (jnp.dot(h_shift, stw0_ref[i], preferred_element_type=f32)
             + jnp.dot(h, stw1_ref[i], preferred_element_type=f32)
             + stb_ref[i:i + 1, :])
        z = jnp.where(z >= 0, z, 0.2 * z)
        h = h + z
    logits = jnp.dot(h, hw_ref[...], preferred_element_type=f32) + hb_ref[...]
    pa_out = jnp.dot(h, hpw_ref[...], preferred_element_type=f32) + hpb_ref[...]
    NA = logits.shape[1]
    out_ref[0, :, :NA] = logits
    out_ref[0, :, NA:] = pa_out


def kernel(atoms, pos_amp, embed_table, pa_w, pa_b, reduce_w, reduce_b,
           stack_w, stack_b, head_atom_w, head_atom_b, head_pa_w, head_pa_b):
    B, T = atoms.shape
    NA, C = embed_table.shape
    dilations = (1, 3, 9, 27, 81, 1)

    idx = atoms.reshape(-1).astype(jnp.int32)
    x = _sc_gather(embed_table, idx).reshape(B, T, C)

    wrx = reduce_w[:, :C, 0].T
    wrpa = reduce_w[:, C:, 0].T
    stw0 = jnp.transpose(stack_w[..., 0], (0, 2, 1))
    stw1 = jnp.transpose(stack_w[..., 1], (0, 2, 1))
    pa_b2 = pa_b.reshape(1, C)
    red_b2 = reduce_b.reshape(1, C)
    hb2 = head_atom_b.reshape(1, NA)
    hpb2 = head_pa_b.reshape(1, 2)

    full = lambda shape: pl.BlockSpec(shape, lambda b: (0,) * len(shape))
    out = pl.pallas_call(
        functools.partial(_tc_body, dilations=dilations),
        grid=(B,),
        in_specs=[
            pl.BlockSpec((1, T, C), lambda b: (b, 0, 0)),
            pl.BlockSpec((1, T, 2), lambda b: (b, 0, 0)),
            full((2, C)),
            full((1, C)),
            full((C, C)),
            full((C, C)),
            full((1, C)),
            full((len(dilations), C, C)),
            full((len(dilations), C, C)),
            full((len(dilations), C)),
            full((C, NA)),
            full((1, NA)),
            full((C, 2)),
            full((1, 2)),
        ],
        out_specs=pl.BlockSpec((1, T, NA + 2), lambda b: (b, 0, 0)),
        out_shape=jax.ShapeDtypeStruct((B, T, NA + 2), jnp.float32),
        compiler_params=pltpu.CompilerParams(
            dimension_semantics=("arbitrary",)),
    )(x, pos_amp, pa_w, pa_b2, wrx, wrpa, red_b2, stw0, stw1, stack_b,
      head_atom_w, hb2, head_pa_w, hpb2)
    return out


# speedup vs baseline: 1.9747x; 1.0020x over previous
import functools

import jax
import jax.numpy as jnp
from jax import lax
from jax.experimental import pallas as pl
from jax.experimental.pallas import tpu as pltpu
from jax.experimental.pallas import tpu_sc as plsc


def _sc_gather(table, idx):
    V, D = table.shape
    N = idx.shape[0]
    info = plsc.get_sparse_core_info()
    NC, NS = info.num_cores, info.num_subcores
    NW = NC * NS
    n_per_w = N // NW
    CHUNK = 256
    n_chunks = n_per_w // CHUNK
    mesh = plsc.VectorSubcoreMesh(core_axis_name="c", subcore_axis_name="s")

    @functools.partial(
        pl.kernel, mesh=mesh,
        out_type=jax.ShapeDtypeStruct((N, D), jnp.float32),
        scratch_types=[
            pltpu.VMEM((CHUNK,), jnp.int32),
            pltpu.VMEM((CHUNK, D), jnp.float32),
            pltpu.VMEM((CHUNK, D), jnp.float32),
            pltpu.SemaphoreType.DMA,
            pltpu.SemaphoreType.DMA,
        ],
    )
    def k(table_hbm, idx_hbm, out_hbm, idx_v, rows_a, rows_b, sem_a, sem_b):
        wid = lax.axis_index("s") * NC + lax.axis_index("c")
        base = wid * n_per_w
        bufs = ((rows_a, sem_a), (rows_b, sem_b))
        for j in range(n_chunks):
            rows_v, sem = bufs[j % 2]
            off = base + j * CHUNK
            pltpu.sync_copy(idx_hbm.at[pl.ds(off, CHUNK)], idx_v)
            pltpu.async_copy(table_hbm.at[idx_v], rows_v, sem).wait()
            pltpu.sync_copy(rows_v, out_hbm.at[pl.ds(off, CHUNK)])

    return k(table, idx)


def _tc_body(x_ref, pa_in_ref, pa_w_ref, pa_b_ref, wrx_ref, wrpa_ref,
             red_b_ref, stw0_ref, stw1_ref, stb_ref, hw_ref, hb_ref,
             hpw_ref, hpb_ref, out_ref, *, dilations):
    T, C = x_ref.shape[1], x_ref.shape[2]
    f32 = jnp.float32
    x = x_ref[0]
    pos_amp = pa_in_ref[0]
    pa = jnp.dot(pos_amp, pa_w_ref[...], preferred_element_type=f32) + pa_b_ref[...]
    h = (jnp.dot(x, wrx_ref[...], preferred_element_type=f32)
         + jnp.dot(pa, wrpa_ref[...], preferred_element_type=f32)
         + red_b_ref[...])
    for i, d in enumerate(dilations):
        h_shift = jnp.concatenate([jnp.zeros((d, C), f32), h[:T - d, :]], axis=0)
        z = (jnp.dot(h_shift, stw0_ref[i], preferred_element_type=f32)
             + jnp.dot(h, stw1_ref[i], preferred_element_type=f32)
             + stb_ref[i:i + 1, :])
        z = jnp.where(z >= 0, z, 0.2 * z)
        h = h + z
    logits = jnp.dot(h.astype(jnp.bfloat16), hw_ref[...],
                     preferred_element_type=f32) + hb_ref[...]
    pa_out = jnp.dot(h, hpw_ref[...], preferred_element_type=f32) + hpb_ref[...]
    NA = logits.shape[1]
    out_ref[0, :, :NA] = logits
    out_ref[0, :, NA:] = pa_out


def kernel(atoms, pos_amp, embed_table, pa_w, pa_b, reduce_w, reduce_b,
           stack_w, stack_b, head_atom_w, head_atom_b, head_pa_w, head_pa_b):
    B, T = atoms.shape
    NA, C = embed_table.shape
    dilations = (1, 3, 9, 27, 81, 1)

    idx = atoms.reshape(-1).astype(jnp.int32)
    x = _sc_gather(embed_table, idx).reshape(B, T, C)

    wrx = reduce_w[:, :C, 0].T
    wrpa = reduce_w[:, C:, 0].T
    stw0 = jnp.transpose(stack_w[..., 0], (0, 2, 1))
    stw1 = jnp.transpose(stack_w[..., 1], (0, 2, 1))
    pa_b2 = pa_b.reshape(1, C)
    red_b2 = reduce_b.reshape(1, C)
    hb2 = head_atom_b.reshape(1, NA)
    hpb2 = head_pa_b.reshape(1, 2)

    full = lambda shape: pl.BlockSpec(shape, lambda b: (0,) * len(shape))
    out = pl.pallas_call(
        functools.partial(_tc_body, dilations=dilations),
        grid=(B,),
        in_specs=[
            pl.BlockSpec((1, T, C), lambda b: (b, 0, 0)),
            pl.BlockSpec((1, T, 2), lambda b: (b, 0, 0)),
            full((2, C)),
            full((1, C)),
            full((C, C)),
            full((C, C)),
            full((1, C)),
            full((len(dilations), C, C)),
            full((len(dilations), C, C)),
            full((len(dilations), C)),
            full((C, NA)),
            full((1, NA)),
            full((C, 2)),
            full((1, 2)),
        ],
        out_specs=pl.BlockSpec((1, T, NA + 2), lambda b: (b, 0, 0)),
        out_shape=jax.ShapeDtypeStruct((B, T, NA + 2), jnp.float32),
        compiler_params=pltpu.CompilerParams(
            dimension_semantics=("parallel",)),
    )(x, pos_amp, pa_w, pa_b2, wrx, wrpa, red_b2, stw0, stw1, stack_b,
      head_atom_w.astype(jnp.bfloat16), hb2, head_pa_w, hpb2)
    return out
